# baseline (speedup 1.0000x reference)
import jax
import jax.numpy as jnp
from jax import lax
from jax.experimental import pallas as pl
from jax.experimental.pallas import tpu as pltpu
from jax.experimental.pallas import tpu_sc as plsc

_NUM_CORES = 2
_NUM_SUBCORES = 16
_NW = _NUM_CORES * _NUM_SUBCORES
_IDXW = 128
_PADW = 128
_LANES = 16
_CHUNK = 256
_BLOCK = 1024


def _make_kernel(n_tokens, d):
    per_w = n_tokens // _NW
    n_blocks = per_w // _BLOCK
    n_chunks = _BLOCK // _CHUNK
    rows_per_chunk = _CHUNK // _IDXW
    mesh = plsc.VectorSubcoreMesh(core_axis_name="c", subcore_axis_name="s")

    def body(rows_hbm, off_hbm, comb_hbm, out_hbm, idx_v, off_v,
             buf_a, buf_b,
             gsem_a, gsem_b, wsem_a, wsem_b):
        wid = lax.axis_index("s") * _NUM_CORES + lax.axis_index("c")
        w_base = wid * per_w

        bufs = (buf_a, buf_b)
        gsems = (gsem_a, gsem_b)
        wsems = (wsem_a, wsem_b)

        def block_body(k, carry):
            base = w_base + k * _BLOCK
            idrow0 = pl.multiple_of(base // _IDXW, 8)
            pltpu.sync_copy(rows_hbm.at[pl.ds(idrow0, 8)], idx_v)
            pltpu.sync_copy(off_hbm.at[pl.ds(idrow0, 8)], off_v)

            ghandles = [None, None]
            whandles = [None, None]

            def fire_gather(p, c):
                ghandles[p] = [
                    pltpu.async_copy(
                        comb_hbm.at[idx_v.at[c * rows_per_chunk + j]],
                        bufs[p].at[pl.ds(j * _IDXW, _IDXW)],
                        gsems[p],
                    )
                    for j in range(rows_per_chunk)
                ]

            fire_gather(0, 0)
            for c in range(n_chunks):
                p = c % 2
                q = (c + 1) % 2
                if c + 1 < n_chunks:
                    if whandles[q] is not None:
                        whandles[q].wait()
                        whandles[q] = None
                    fire_gather(q, c + 1)
                for h in ghandles[p]:
                    h.wait()

                def grp_body(g, carry2):
                    r = c * rows_per_chunk + g // (_IDXW // _LANES)
                    gco = (g % (_IDXW // _LANES)) * _LANES
                    offv = off_v[r, pl.ds(gco, _LANES)]
                    j0 = g * _LANES
                    for jj in range(_LANES):
                        off = offv[jj]
                        j = j0 + jj
                        for k2 in range(d // _LANES):
                            bufs[p][j, pl.ds(k2 * _LANES, _LANES)] = (
                                bufs[p][j, pl.ds(off + k2 * _LANES, _LANES)])
                    return carry2

                lax.fori_loop(0, _CHUNK // _LANES, grp_body, 0)

                orow = pl.multiple_of(base + c * _CHUNK, 8)
                whandles[p] = pltpu.async_copy(
                    bufs[p], out_hbm.at[pl.ds(orow, _CHUNK)], wsems[p])
            for p in (0, 1):
                if whandles[p] is not None:
                    whandles[p].wait()
            return carry

        lax.fori_loop(0, n_blocks, block_body, 0)

    return pl.kernel(
        body,
        out_type=jax.ShapeDtypeStruct((n_tokens, _PADW), jnp.float32),
        mesh=mesh,
        scratch_types=[
            pltpu.VMEM((8, _IDXW), jnp.int32),
            pltpu.VMEM((8, _IDXW), jnp.int32),
            pltpu.VMEM((_CHUNK, _PADW), jnp.float32),
            pltpu.VMEM((_CHUNK, _PADW), jnp.float32),
            pltpu.SemaphoreType.DMA,
            pltpu.SemaphoreType.DMA,
            pltpu.SemaphoreType.DMA,
            pltpu.SemaphoreType.DMA,
        ],
    )


@jax.jit
def kernel(input_ids, image_embeds, table):
    b, s = input_ids.shape
    d = table.shape[1]
    ids = input_ids.reshape(-1, _IDXW).astype(jnp.int32)
    rows = ids // 2
    offs = (ids & 1) * d
    combined = jnp.concatenate([table, image_embeds.astype(table.dtype)], axis=0)
    combined = combined.reshape(-1, _PADW)
    k = _make_kernel(ids.size, d)
    out = k(rows, offs, combined)
    return out[:, :d].reshape(b, s, d)


# speedup vs baseline: 1.5179x; 1.5179x over previous
import jax
import jax.numpy as jnp
from jax import lax
from jax.experimental import pallas as pl
from jax.experimental.pallas import tpu as pltpu
from jax.experimental.pallas import tpu_sc as plsc

_NUM_CORES = 2
_NUM_SUBCORES = 16
_NW = _NUM_CORES * _NUM_SUBCORES
_IDXW = 128
_PADW = 128
_CHUNK = 256
_BLOCK = 1024


def _make_kernel(n_tokens, d):
    per_w = n_tokens // _NW
    n_blocks = per_w // _BLOCK
    mesh = plsc.VectorSubcoreMesh(core_axis_name="c", subcore_axis_name="s")

    def body(ids_hbm, comb_hbm, out_hbm, idx_v,
             buf_a, buf_b, buf_c,
             gsem_a, gsem_b, gsem_c, wsem_a, wsem_b, wsem_c):
        wid = lax.axis_index("s") * _NUM_CORES + lax.axis_index("c")
        w_base = wid * per_w

        bufs = (buf_a, buf_b, buf_c)
        gsems = (gsem_a, gsem_b, gsem_c)
        wsems = (wsem_a, wsem_b, wsem_c)
        n_chunks = _BLOCK // _CHUNK
        rows_per_chunk = _CHUNK // _IDXW

        def block_body(k, carry):
            base = w_base + k * _BLOCK
            idrow0 = pl.multiple_of(base // _IDXW, 8)
            pltpu.sync_copy(ids_hbm.at[pl.ds(idrow0, 8)], idx_v)

            ghandles = [None, None, None]
            whandles = [None, None, None]

            def start_gather(p, c):
                if whandles[p] is not None:
                    whandles[p].wait()
                    whandles[p] = None
                rows = [c * rows_per_chunk + j for j in range(rows_per_chunk)]
                ghandles[p] = [
                    pltpu.async_copy(
                        comb_hbm.at[idx_v.at[rows[j]]],
                        bufs[p].at[pl.ds(j * _IDXW, _IDXW)],
                        gsems[p],
                    )
                    for j in range(rows_per_chunk)
                ]

            def start_write(p, c):
                for h in ghandles[p]:
                    h.wait()
                ghandles[p] = None
                orow = pl.multiple_of(base + c * _CHUNK, 8)
                whandles[p] = pltpu.async_copy(
                    bufs[p], out_hbm.at[pl.ds(orow, _CHUNK)], wsems[p])

            for c in range(n_chunks):
                start_gather(c % 3, c)
                if c >= 1:
                    start_write((c - 1) % 3, c - 1)
            start_write((n_chunks - 1) % 3, n_chunks - 1)
            for p in range(3):
                if whandles[p] is not None:
                    whandles[p].wait()
            return carry

        lax.fori_loop(0, n_blocks, block_body, 0)

    return pl.kernel(
        body,
        out_type=jax.ShapeDtypeStruct((n_tokens, _PADW), jnp.float32),
        mesh=mesh,
        scratch_types=[
            pltpu.VMEM((8, _IDXW), jnp.int32),
            pltpu.VMEM((_CHUNK, _PADW), jnp.float32),
            pltpu.VMEM((_CHUNK, _PADW), jnp.float32),
            pltpu.VMEM((_CHUNK, _PADW), jnp.float32),
            pltpu.SemaphoreType.DMA,
            pltpu.SemaphoreType.DMA,
            pltpu.SemaphoreType.DMA,
            pltpu.SemaphoreType.DMA,
            pltpu.SemaphoreType.DMA,
            pltpu.SemaphoreType.DMA,
        ],
    )


@jax.jit
def kernel(input_ids, image_embeds, table):
    b, s = input_ids.shape
    d = table.shape[1]
    ids = input_ids.reshape(-1, _IDXW).astype(jnp.int32)
    combined = jnp.concatenate([table, image_embeds.astype(table.dtype)], axis=0)
    combined = jnp.pad(combined, ((0, 0), (0, _PADW - d)))
    k = _make_kernel(ids.size, d)
    out = k(ids, combined)
    return out[:, :d].reshape(b, s, d)


# speedup vs baseline: 1.5366x; 1.0123x over previous
import jax
import jax.numpy as jnp
from jax import lax
from jax.experimental import pallas as pl
from jax.experimental.pallas import tpu as pltpu
from jax.experimental.pallas import tpu_sc as plsc

_NUM_CORES = 2
_NUM_SUBCORES = 16
_NW = _NUM_CORES * _NUM_SUBCORES
_IDXW = 128
_PADW = 128
_CHUNK = 256
_BLOCK = 2048


def _make_kernel(n_tokens, d):
    per_w = n_tokens // _NW
    n_blocks = per_w // _BLOCK
    mesh = plsc.VectorSubcoreMesh(core_axis_name="c", subcore_axis_name="s")

    def body(ids_hbm, comb_hbm, out_hbm, idx_v,
             buf_a, buf_b, buf_c,
             gsem_a, gsem_b, gsem_c, wsem_a, wsem_b, wsem_c):
        wid = lax.axis_index("s") * _NUM_CORES + lax.axis_index("c")
        w_base = wid * per_w

        bufs = (buf_a, buf_b, buf_c)
        gsems = (gsem_a, gsem_b, gsem_c)
        wsems = (wsem_a, wsem_b, wsem_c)
        n_chunks = _BLOCK // _CHUNK
        rows_per_chunk = _CHUNK // _IDXW

        def block_body(k, carry):
            base = w_base + k * _BLOCK
            idrow0 = pl.multiple_of(base // _IDXW, 8)
            pltpu.sync_copy(
                ids_hbm.at[pl.ds(idrow0, _BLOCK // _IDXW)], idx_v)

            ghandles = [None, None, None]
            whandles = [None, None, None]

            def start_gather(p, c):
                if whandles[p] is not None:
                    whandles[p].wait()
                    whandles[p] = None
                rows = [c * rows_per_chunk + j for j in range(rows_per_chunk)]
                ghandles[p] = [
                    pltpu.async_copy(
                        comb_hbm.at[idx_v.at[rows[j]]],
                        bufs[p].at[pl.ds(j * _IDXW, _IDXW)],
                        gsems[p],
                    )
                    for j in range(rows_per_chunk)
                ]

            def start_write(p, c):
                for h in ghandles[p]:
                    h.wait()
                ghandles[p] = None
                orow = pl.multiple_of(base + c * _CHUNK, 8)
                whandles[p] = pltpu.async_copy(
                    bufs[p], out_hbm.at[pl.ds(orow, _CHUNK)], wsems[p])

            for c in range(n_chunks):
                start_gather(c % 3, c)
                if c >= 1:
                    start_write((c - 1) % 3, c - 1)
            start_write((n_chunks - 1) % 3, n_chunks - 1)
            for p in range(3):
                if whandles[p] is not None:
                    whandles[p].wait()
            return carry

        lax.fori_loop(0, n_blocks, block_body, 0)

    return pl.kernel(
        body,
        out_type=jax.ShapeDtypeStruct((n_tokens, _PADW), jnp.float32),
        mesh=mesh,
        scratch_types=[
            pltpu.VMEM((_BLOCK // _IDXW, _IDXW), jnp.int32),
            pltpu.VMEM((_CHUNK, _PADW), jnp.float32),
            pltpu.VMEM((_CHUNK, _PADW), jnp.float32),
            pltpu.VMEM((_CHUNK, _PADW), jnp.float32),
            pltpu.SemaphoreType.DMA,
            pltpu.SemaphoreType.DMA,
            pltpu.SemaphoreType.DMA,
            pltpu.SemaphoreType.DMA,
            pltpu.SemaphoreType.DMA,
            pltpu.SemaphoreType.DMA,
        ],
    )


@jax.jit
def kernel(input_ids, image_embeds, table):
    b, s = input_ids.shape
    d = table.shape[1]
    ids = input_ids.reshape(-1, _IDXW).astype(jnp.int32)
    combined = jnp.concatenate([table, image_embeds.astype(table.dtype)], axis=0)
    combined = jnp.pad(combined, ((0, 0), (0, _PADW - d)))
    k = _make_kernel(ids.size, d)
    out = k(ids, combined)
    return out[:, :d].reshape(b, s, d)
